# Initial kernel scaffold; baseline (speedup 1.0000x reference)
import jax
import jax.numpy as jnp
from jax.experimental import pallas as pl


def kernel(x, weight, expert_bias):
    raise NotImplementedError("write your pallas kernel here")


# speedup vs baseline: 1.4811x; 1.4811x over previous
import functools

import jax
import jax.numpy as jnp
from jax import lax
from jax.experimental import pallas as pl
from jax.experimental.pallas import tpu as pltpu
from jax.experimental.pallas import tpu_sc as plsc

TOPK = 2
NEXP = 8
HID = 2048
ALPHA = 0.01
BSZ = 4
SEQ = 8192
NTOK = BSZ * SEQ
NTILES = 16
TPT = NTOK // NTILES
LANES = 16
GROUPS = TPT // LANES
TILES_PER_BATCH = SEQ // TPT


def _logits_body(wt_ref, x_ref, o_ref):
    xb = x_ref[...].astype(jnp.bfloat16)
    wb = wt_ref[...].astype(jnp.bfloat16)
    acc = lax.dot_general(
        xb, wb,
        dimension_numbers=(((1,), (0,)), ((), ())),
        preferred_element_type=jnp.float32,
    )
    o_ref[...] = acc.T


def _compute_logits(x2, wt):
    tblk = 512
    return pl.pallas_call(
        _logits_body,
        grid=(NTOK // tblk,),
        in_specs=[
            pl.BlockSpec((HID, NEXP), lambda i: (0, 0)),
            pl.BlockSpec((tblk, HID), lambda i: (i, 0)),
        ],
        out_specs=pl.BlockSpec((NEXP, tblk), lambda i: (0, i)),
        out_shape=jax.ShapeDtypeStruct((NEXP, NTOK), jnp.float32),
    )(wt, x2)


def _router_body(logits_hbm, bias_hbm, idx_out, w_out, loss_out,
                 lg_v, bias_v, idxbuf_v, wbuf_v, stage_v, stage2_v, comb_v, shared_sp):
    wid = lax.axis_index("s")
    base = wid * TPT

    for e in range(NEXP):
        pltpu.sync_copy(logits_hbm.at[e, pl.ds(base, TPT)], lg_v.at[e])
    pltpu.sync_copy(bias_hbm, bias_v)

    bias_vec = bias_v[...]
    bias_s = [bias_vec[e] for e in range(NEXP)]
    iot = lax.iota(jnp.int32, LANES)
    zero_f = jnp.zeros((LANES,), jnp.float32)
    one_f = jnp.ones((LANES,), jnp.float32)

    def grp(g, carry):
        faccs, paccs = carry
        off = pl.multiple_of(g * LANES, LANES)
        ls = [lg_v[e, pl.ds(off, LANES)] for e in range(NEXP)]
        ss = [1.0 / (1.0 + jnp.exp(-l)) for l in ls]
        bs = [ls[e] + bias_s[e] for e in range(NEXP)]

        m1 = bs[0]
        i1 = jnp.zeros((LANES,), jnp.int32)
        s1 = ss[0]
        m2 = jnp.full((LANES,), -jnp.inf, jnp.float32)
        i2 = jnp.zeros((LANES,), jnp.int32)
        s2 = zero_f
        for e in range(1, NEXP):
            ev = jnp.full((LANES,), e, jnp.int32)
            gt1 = bs[e] > m1
            gt2 = bs[e] > m2
            m2n = jnp.where(gt1, m1, jnp.where(gt2, bs[e], m2))
            i2n = jnp.where(gt1, i1, jnp.where(gt2, ev, i2))
            s2n = jnp.where(gt1, s1, jnp.where(gt2, ss[e], s2))
            m1 = jnp.where(gt1, bs[e], m1)
            i1 = jnp.where(gt1, ev, i1)
            s1 = jnp.where(gt1, ss[e], s1)
            m2, i2, s2 = m2n, i2n, s2n

        den = s1 + s2 + jnp.float32(1e-10)
        w1 = s1 / den
        w2 = s2 / den

        ssum = ss[0]
        for e in range(1, NEXP):
            ssum = ssum + ss[e]
        inv = 1.0 / (ssum + jnp.float32(1e-10))

        new_f = []
        new_p = []
        for e in range(NEXP):
            ev = jnp.full((LANES,), e, jnp.int32)
            cnt = (jnp.where(i1 == ev, one_f, zero_f)
                   + jnp.where(i2 == ev, one_f, zero_f))
            new_f.append(faccs[e] + cnt)
            new_p.append(paccs[e] + ss[e] * inv)

        idxbuf_v[0, pl.ds(off, LANES)] = i1
        idxbuf_v[1, pl.ds(off, LANES)] = i2
        wbuf_v[0, pl.ds(off, LANES)] = w1
        wbuf_v[1, pl.ds(off, LANES)] = w2
        return (new_f, new_p)

    init = ([zero_f] * NEXP, [zero_f] * NEXP)
    faccs, paccs = lax.fori_loop(0, GROUPS, grp, init)

    for r in range(TOPK):
        pltpu.sync_copy(idxbuf_v.at[r], idx_out.at[r, pl.ds(base, TPT)])
        pltpu.sync_copy(wbuf_v.at[r], w_out.at[r, pl.ds(base, TPT)])

    fpart = jnp.zeros((LANES,), jnp.float32)
    ppart = jnp.zeros((LANES,), jnp.float32)
    for e in range(NEXP):
        fpart = jnp.where(iot == e, jnp.sum(faccs[e]), fpart)
        ppart = jnp.where(iot == e, jnp.sum(paccs[e]), ppart)
    stage_v[...] = fpart
    stage2_v[...] = ppart
    pltpu.sync_copy(stage_v, shared_sp.at[pl.ds(wid * LANES, LANES)])
    pltpu.sync_copy(stage2_v,
                    shared_sp.at[pl.ds((NTILES + wid) * LANES, LANES)])
    plsc.subcore_barrier()

    @pl.when(wid == 0)
    def _():
        pltpu.sync_copy(shared_sp, comb_v)
        acc = jnp.float32(0.0)
        for b in range(BSZ):
            r0 = b * TILES_PER_BATCH
            fvec = comb_v[pl.ds(r0 * LANES, LANES)]
            pvec = comb_v[pl.ds((NTILES + r0) * LANES, LANES)]
            for j in range(1, TILES_PER_BATCH):
                fvec = fvec + comb_v[pl.ds((r0 + j) * LANES, LANES)]
                pvec = pvec + comb_v[pl.ds((NTILES + r0 + j) * LANES, LANES)]
            acc = acc + jnp.sum(fvec * pvec)
        loss = acc * jnp.float32(ALPHA / (BSZ * TOPK * SEQ * SEQ))
        stage_v[...] = jnp.where(iot == 0, loss, jnp.float32(0.0))
        pltpu.sync_copy(stage_v, loss_out)


@functools.partial(
    pl.kernel,
    out_type=[
        jax.ShapeDtypeStruct((TOPK, NTOK), jnp.int32),
        jax.ShapeDtypeStruct((TOPK, NTOK), jnp.float32),
        jax.ShapeDtypeStruct((LANES,), jnp.float32),
    ],
    mesh=plsc.VectorSubcoreMesh(
        core_axis_name="c", subcore_axis_name="s", num_cores=1),
    compiler_params=pltpu.CompilerParams(needs_layout_passes=False),
    scratch_types=[
        pltpu.VMEM((NEXP, TPT), jnp.float32),
        pltpu.VMEM((LANES,), jnp.float32),
        pltpu.VMEM((TOPK, TPT), jnp.int32),
        pltpu.VMEM((TOPK, TPT), jnp.float32),
        pltpu.VMEM((LANES,), jnp.float32),
        pltpu.VMEM((LANES,), jnp.float32),
        pltpu.VMEM((2 * NTILES * LANES,), jnp.float32),
        pltpu.VMEM_SHARED((2 * NTILES * LANES,), jnp.float32),
    ],
)
def _router(logits_hbm, bias_hbm, idx_out, w_out, loss_out,
            lg_v, bias_v, idxbuf_v, wbuf_v, stage_v, stage2_v, comb_v, shared_sp):
    _router_body(logits_hbm, bias_hbm, idx_out, w_out, loss_out,
                 lg_v, bias_v, idxbuf_v, wbuf_v, stage_v, stage2_v, comb_v, shared_sp)


def kernel(x, weight, expert_bias):
    x2 = x.reshape(NTOK, HID)
    logits_t = _compute_logits(x2, weight.T)
    bias16 = jnp.pad(expert_bias, (0, LANES - NEXP))
    idx_rows, w_rows, loss_vec = _router(logits_t, bias16)
    topk_indices = idx_rows.T
    topk_weights = w_rows.T
    return topk_indices, topk_weights, loss_vec[0]


# speedup vs baseline: 1.7169x; 1.1592x over previous
import functools

import jax
import jax.numpy as jnp
from jax import lax
from jax.experimental import pallas as pl
from jax.experimental.pallas import tpu as pltpu
from jax.experimental.pallas import tpu_sc as plsc

TOPK = 2
NEXP = 8
HID = 2048
ALPHA = 0.01
BSZ = 4
SEQ = 8192
NTOK = BSZ * SEQ
NTILES = 16
TPT = NTOK // NTILES
LANES = 16
GROUPS = TPT // LANES
TILES_PER_BATCH = SEQ // TPT


def _logits_body(wt_ref, x_ref, o_ref):
    xb = x_ref[...].astype(jnp.bfloat16)
    wb = wt_ref[...].astype(jnp.bfloat16)
    acc = lax.dot_general(
        xb, wb,
        dimension_numbers=(((1,), (0,)), ((), ())),
        preferred_element_type=jnp.float32,
    )
    o_ref[...] = acc.T


def _compute_logits(x2, wt):
    tblk = 1024
    return pl.pallas_call(
        _logits_body,
        grid=(NTOK // tblk,),
        in_specs=[
            pl.BlockSpec((HID, NEXP), lambda i: (0, 0)),
            pl.BlockSpec((tblk, HID), lambda i: (i, 0)),
        ],
        out_specs=pl.BlockSpec((NEXP, tblk), lambda i: (0, i)),
        out_shape=jax.ShapeDtypeStruct((NEXP, NTOK), jnp.float32),
    )(wt, x2)


def _router_body(logits_hbm, bias_hbm, idx_out, w_out, loss_out,
                 lg_v, bias_v, idxbuf_v, wbuf_v, stage_v, stage2_v, comb_v, shared_sp):
    wid = lax.axis_index("s")
    base = wid * TPT

    for e in range(NEXP):
        pltpu.sync_copy(logits_hbm.at[e, pl.ds(base, TPT)], lg_v.at[e])
    pltpu.sync_copy(bias_hbm, bias_v)

    bias_vec = bias_v[...]
    bias_s = [bias_vec[e] for e in range(NEXP)]
    iot = lax.iota(jnp.int32, LANES)
    zero_f = jnp.zeros((LANES,), jnp.float32)
    one_f = jnp.ones((LANES,), jnp.float32)

    def grp(g, carry):
        faccs, paccs = carry
        off = pl.multiple_of(g * LANES, LANES)
        ls = [lg_v[e, pl.ds(off, LANES)] for e in range(NEXP)]
        ss = [1.0 / (1.0 + jnp.exp(-l)) for l in ls]
        bs = [ls[e] + bias_s[e] for e in range(NEXP)]

        m1 = bs[0]
        i1 = jnp.zeros((LANES,), jnp.int32)
        s1 = ss[0]
        m2 = jnp.full((LANES,), -jnp.inf, jnp.float32)
        i2 = jnp.zeros((LANES,), jnp.int32)
        s2 = zero_f
        for e in range(1, NEXP):
            ev = jnp.full((LANES,), e, jnp.int32)
            gt1 = bs[e] > m1
            gt2 = bs[e] > m2
            m2n = jnp.where(gt1, m1, jnp.where(gt2, bs[e], m2))
            i2n = jnp.where(gt1, i1, jnp.where(gt2, ev, i2))
            s2n = jnp.where(gt1, s1, jnp.where(gt2, ss[e], s2))
            m1 = jnp.where(gt1, bs[e], m1)
            i1 = jnp.where(gt1, ev, i1)
            s1 = jnp.where(gt1, ss[e], s1)
            m2, i2, s2 = m2n, i2n, s2n

        den = s1 + s2 + jnp.float32(1e-10)
        w1 = s1 / den
        w2 = s2 / den

        ssum = ss[0]
        for e in range(1, NEXP):
            ssum = ssum + ss[e]
        inv = 1.0 / (ssum + jnp.float32(1e-10))

        new_f = []
        new_p = []
        for e in range(NEXP):
            ev = jnp.full((LANES,), e, jnp.int32)
            cnt = (jnp.where(i1 == ev, one_f, zero_f)
                   + jnp.where(i2 == ev, one_f, zero_f))
            new_f.append(faccs[e] + cnt)
            new_p.append(paccs[e] + ss[e] * inv)

        idxbuf_v[0, pl.ds(off, LANES)] = i1
        idxbuf_v[1, pl.ds(off, LANES)] = i2
        wbuf_v[0, pl.ds(off, LANES)] = w1
        wbuf_v[1, pl.ds(off, LANES)] = w2
        return (new_f, new_p)

    init = ([zero_f] * NEXP, [zero_f] * NEXP)
    faccs, paccs = lax.fori_loop(0, GROUPS, grp, init)

    for r in range(TOPK):
        pltpu.sync_copy(idxbuf_v.at[r], idx_out.at[r, pl.ds(base, TPT)])
        pltpu.sync_copy(wbuf_v.at[r], w_out.at[r, pl.ds(base, TPT)])

    fpart = jnp.zeros((LANES,), jnp.float32)
    ppart = jnp.zeros((LANES,), jnp.float32)
    for e in range(NEXP):
        fpart = jnp.where(iot == e, jnp.sum(faccs[e]), fpart)
        ppart = jnp.where(iot == e, jnp.sum(paccs[e]), ppart)
    stage_v[...] = fpart
    stage2_v[...] = ppart
    pltpu.sync_copy(stage_v, shared_sp.at[pl.ds(wid * LANES, LANES)])
    pltpu.sync_copy(stage2_v,
                    shared_sp.at[pl.ds((NTILES + wid) * LANES, LANES)])
    plsc.subcore_barrier()

    @pl.when(wid == 0)
    def _():
        pltpu.sync_copy(shared_sp, comb_v)
        acc = jnp.float32(0.0)
        for b in range(BSZ):
            r0 = b * TILES_PER_BATCH
            fvec = comb_v[pl.ds(r0 * LANES, LANES)]
            pvec = comb_v[pl.ds((NTILES + r0) * LANES, LANES)]
            for j in range(1, TILES_PER_BATCH):
                fvec = fvec + comb_v[pl.ds((r0 + j) * LANES, LANES)]
                pvec = pvec + comb_v[pl.ds((NTILES + r0 + j) * LANES, LANES)]
            acc = acc + jnp.sum(fvec * pvec)
        loss = acc * jnp.float32(ALPHA / (BSZ * TOPK * SEQ * SEQ))
        stage_v[...] = jnp.where(iot == 0, loss, jnp.float32(0.0))
        pltpu.sync_copy(stage_v, loss_out)


@functools.partial(
    pl.kernel,
    out_type=[
        jax.ShapeDtypeStruct((TOPK, NTOK), jnp.int32),
        jax.ShapeDtypeStruct((TOPK, NTOK), jnp.float32),
        jax.ShapeDtypeStruct((LANES,), jnp.float32),
    ],
    mesh=plsc.VectorSubcoreMesh(
        core_axis_name="c", subcore_axis_name="s", num_cores=1),
    compiler_params=pltpu.CompilerParams(needs_layout_passes=False),
    scratch_types=[
        pltpu.VMEM((NEXP, TPT), jnp.float32),
        pltpu.VMEM((LANES,), jnp.float32),
        pltpu.VMEM((TOPK, TPT), jnp.int32),
        pltpu.VMEM((TOPK, TPT), jnp.float32),
        pltpu.VMEM((LANES,), jnp.float32),
        pltpu.VMEM((LANES,), jnp.float32),
        pltpu.VMEM((2 * NTILES * LANES,), jnp.float32),
        pltpu.VMEM_SHARED((2 * NTILES * LANES,), jnp.float32),
    ],
)
def _router(logits_hbm, bias_hbm, idx_out, w_out, loss_out,
            lg_v, bias_v, idxbuf_v, wbuf_v, stage_v, stage2_v, comb_v, shared_sp):
    _router_body(logits_hbm, bias_hbm, idx_out, w_out, loss_out,
                 lg_v, bias_v, idxbuf_v, wbuf_v, stage_v, stage2_v, comb_v, shared_sp)


def kernel(x, weight, expert_bias):
    x2 = x.reshape(NTOK, HID)
    logits_t = _compute_logits(x2, weight.T)
    bias16 = jnp.pad(expert_bias, (0, LANES - NEXP))
    idx_rows, w_rows, loss_vec = _router(logits_t, bias16)
    topk_indices = idx_rows.T
    topk_weights = w_rows.T
    return topk_indices, topk_weights, loss_vec[0]
